# baseline (speedup 1.0000x reference)
import functools

import jax
import jax.numpy as jnp
from jax.experimental import pallas as pl
from jax.experimental.pallas import tpu as pltpu
from jax.experimental.pallas import tpu_sc as plsc

_D = 2048
_N_TOK = 8192
_TOTAL = _D * _D * 2 + _D * 2
_K = max(1, int(0.05 * _TOTAL))
_TOPBIT = -2147483648

_NC = 2
_NS = 16
_NW = _NC * _NS
_WPS = (_D * _D) // _NW
_CH = 8192
_NCHS = _WPS // _CH
_BW = (2 * _D) // _NW
_NBIN = 4096
_HSLOTS = _NBIN * 16


def _sc_hist_body(shh_ref, sout_ref, sbhh_ref, sbout_ref, out_ref,
                  buf_ref, hist_ref):
    wid = jax.lax.axis_index("s") * _NC + jax.lax.axis_index("c")
    lane = jax.lax.iota(jnp.int32, 16)
    ones = jnp.ones((16,), jnp.int32)

    def zero(i, _):
        hist_ref[pl.ds(i * 16, 16)] = jnp.zeros((16,), jnp.int32)
        return 0

    jax.lax.fori_loop(0, _HSLOTS // 16, zero, 0)

    def eat_block(n):
        def body(i, _):
            v = buf_ref[pl.ds(i * 16, 16)]
            b = jax.lax.bitcast_convert_type(v, jnp.int32)
            u = jnp.where(b >= 0, b ^ jnp.int32(_TOPBIT), jnp.bitwise_not(b))
            binid = jax.lax.shift_right_logical(u, 20)
            plsc.addupdate_scatter(hist_ref, [(binid << 4) | lane], ones)
            return 0

        jax.lax.fori_loop(0, n // 16, body, 0)

    def chunks(src_ref):
        def per_chunk(c, _):
            pltpu.sync_copy(src_ref.at[pl.ds(wid * _WPS + c * _CH, _CH)],
                            buf_ref)
            eat_block(_CH)
            return 0

        jax.lax.fori_loop(0, _NCHS, per_chunk, 0)

    chunks(shh_ref)
    chunks(sout_ref)
    for bias_ref in (sbhh_ref, sbout_ref):
        pltpu.sync_copy(bias_ref.at[pl.ds(wid * (_BW // 2), _BW // 2)],
                        buf_ref.at[pl.ds(0, _BW // 2)])
        eat_block(_BW // 2)

    pltpu.sync_copy(hist_ref, out_ref.at[wid])


def _sc_hist(s_hh, s_bhh, s_out, s_bout):
    mesh = plsc.VectorSubcoreMesh(core_axis_name="c", subcore_axis_name="s")
    fn = pl.kernel(
        _sc_hist_body,
        out_type=jax.ShapeDtypeStruct((_NW, _HSLOTS), jnp.int32),
        scratch_types=[pltpu.VMEM((_CH,), jnp.float32),
                       pltpu.VMEM((_HSLOTS,), jnp.int32)],
        mesh=mesh,
        compiler_params=pltpu.CompilerParams(needs_layout_passes=False),
    )
    return fn(s_hh.reshape(_D * _D), s_out.reshape(_D * _D),
              s_bhh, s_bout)


def _finish_body(hist_ref, s_hh_ref, s_bhh_ref, s_out_ref, s_bout_ref,
                 bits_ref):
    acc = hist_ref[pl.ds(0, 512), :]
    for w in range(1, _NW):
        acc = acc + hist_ref[pl.ds(w * 512, 512), :]
    row_i = jax.lax.broadcasted_iota(jnp.int32, (512, 128), 0)
    lane_i = jax.lax.broadcasted_iota(jnp.int32, (512, 128), 1)
    binmat = row_i * 8 + lane_i // 16

    def binstep(i, pb):
        cand = pb | (jnp.int32(1) << (11 - i))
        cnt = jnp.sum(jnp.where(binmat >= cand, acc, 0))
        return jnp.where(cnt >= _K, cand, pb)

    bstar = jax.lax.fori_loop(0, 12, binstep, jnp.int32(0))

    def count_ge(c):
        n = jnp.sum((s_hh_ref[...] >= c).astype(jnp.int32))
        n += jnp.sum((s_out_ref[...] >= c).astype(jnp.int32))
        n += jnp.sum((s_bhh_ref[...] >= c).astype(jnp.int32))
        n += jnp.sum((s_bout_ref[...] >= c).astype(jnp.int32))
        return n

    def _pat_to_f32(p):
        b = jnp.where(p < 0, p ^ jnp.int32(_TOPBIT), jnp.bitwise_not(p))
        return jax.lax.bitcast_convert_type(
            jnp.full((1, 1), b, jnp.int32), jnp.float32)

    def step(i, p):
        shift = 30 - 2 * i
        hits = jnp.int32(0)
        for d in (1, 2, 3):
            cand = p | (jnp.int32(d) << shift)
            cnt = count_ge(_pat_to_f32(cand))
            hits += (cnt >= _K).astype(jnp.int32)
        return p | (hits << shift)

    p = jax.lax.fori_loop(6, 16, step, bstar << 20)
    bits_ref[0, 0] = jnp.where(p < 0, p ^ jnp.int32(_TOPBIT),
                               jnp.bitwise_not(p))


def _select_finish(hists, s_hh, s_bhh, s_out, s_bout):
    vmem = lambda: pl.BlockSpec(memory_space=pltpu.VMEM)
    return pl.pallas_call(
        _finish_body,
        in_specs=[vmem(), vmem(), vmem(), vmem(), vmem()],
        out_specs=pl.BlockSpec(memory_space=pltpu.SMEM),
        out_shape=jax.ShapeDtypeStruct((1, 1), jnp.int32),
    )(hists.reshape(_NW * 512, 128), s_hh, s_bhh.reshape(8, _D // 8),
      s_out, s_bout.reshape(8, _D // 8))


_NCHUNK = 8
_ROWS = _D // _NCHUNK


def _mask_body(thr_ref, whh_ref, shh_ref, wout_ref, sout_ref,
               bhh_ref, sbhh_ref, bout_ref, sbout_ref,
               wmhh_ref, wmout_ref, bmhh_ref, bmout_ref):
    thr = thr_ref[0, 0]
    wmhh_ref[...] = (whh_ref[...] * (shh_ref[...] >= thr)).astype(jnp.bfloat16)
    wmout_ref[...] = (wout_ref[...] * (sout_ref[...] >= thr)
                      ).astype(jnp.bfloat16)
    bmhh_ref[...] = bhh_ref[...] * (sbhh_ref[...] >= thr).astype(jnp.float32)
    bmout_ref[...] = bout_ref[...] * (sbout_ref[...] >= thr).astype(jnp.float32)


def _mask_weights(thr, whh, shh, wout, sout, bhh, sbhh, bout, sbout):
    big = pl.BlockSpec((_ROWS, _D), lambda c: (c, 0))
    vec = pl.BlockSpec((1, _D), lambda c: (0, 0))
    return pl.pallas_call(
        _mask_body,
        grid=(_NCHUNK,),
        in_specs=[pl.BlockSpec(memory_space=pltpu.SMEM),
                  big, big, big, big, vec, vec, vec, vec],
        out_specs=[big, big, vec, vec],
        out_shape=[jax.ShapeDtypeStruct((_D, _D), jnp.bfloat16),
                   jax.ShapeDtypeStruct((_D, _D), jnp.bfloat16),
                   jax.ShapeDtypeStruct((1, _D), jnp.float32),
                   jax.ShapeDtypeStruct((1, _D), jnp.float32)],
    )(thr, whh, shh, wout, sout, bhh, sbhh, bout, sbout)


_TILE_M = 512


def _dot_nt(a, b):
    return jax.lax.dot_general(a, b, (((1,), (1,)), ((), ())),
                               preferred_element_type=jnp.float32)


def _fwd1_body(x_ref, wih_ref, h_ref, wscr_ref):
    @pl.when(pl.program_id(0) == 0)
    def _():
        wscr_ref[...] = wih_ref[...].astype(jnp.bfloat16)

    h_ref[...] = jnp.tanh(_dot_nt(x_ref[...].astype(jnp.bfloat16),
                                  wscr_ref[...])).astype(jnp.bfloat16)


def _fwd1(x, wih):
    xspec = pl.BlockSpec((_TILE_M, _D), lambda m: (m, 0))
    return pl.pallas_call(
        _fwd1_body,
        grid=(_N_TOK // _TILE_M,),
        in_specs=[xspec, pl.BlockSpec((_D, _D), lambda m: (0, 0))],
        out_specs=xspec,
        out_shape=jax.ShapeDtypeStruct((_N_TOK, _D), jnp.bfloat16),
        scratch_shapes=[pltpu.VMEM((_D, _D), jnp.bfloat16)],
    )(x, wih)


def _fwd23_body(h_ref, wmhh_ref, bmhh_ref, wmout_ref, bmout_ref, out_ref):
    h2 = jnp.tanh(_dot_nt(h_ref[...], wmhh_ref[...]) + bmhh_ref[...])
    out_ref[...] = _dot_nt(h2.astype(jnp.bfloat16), wmout_ref[...]) \
        + bmout_ref[...]


def _fwd23(h, wmhh, bmhh, wmout, bmout):
    xspec = pl.BlockSpec((_TILE_M, _D), lambda m: (m, 0))
    wspec = pl.BlockSpec((_D, _D), lambda m: (0, 0))
    vec = pl.BlockSpec((1, _D), lambda m: (0, 0))
    return pl.pallas_call(
        _fwd23_body,
        grid=(_N_TOK // _TILE_M,),
        in_specs=[xspec, wspec, vec, wspec, vec],
        out_specs=xspec,
        out_shape=jax.ShapeDtypeStruct((_N_TOK, _D), jnp.float32),
    )(h, wmhh, bmhh, wmout, bmout)


def kernel(x, W_ih, W_hh, b_hh, W_out, b_out, s_hh, s_b_hh, s_out, s_b_out):
    hists = _sc_hist(s_hh, s_b_hh, s_out, s_b_out)
    h = _fwd1(x, W_ih)
    bits = _select_finish(hists, s_hh, s_b_hh, s_out, s_b_out)
    thr = jax.lax.bitcast_convert_type(bits, jnp.float32)
    wmhh, wmout, bmhh, bmout = _mask_weights(
        thr, W_hh, s_hh, W_out, s_out,
        b_hh.reshape(1, _D), s_b_hh.reshape(1, _D),
        b_out.reshape(1, _D), s_b_out.reshape(1, _D))
    return _fwd23(h, wmhh, bmhh, wmout, bmout)


# speedup vs baseline: 1.0430x; 1.0430x over previous
import functools

import jax
import jax.numpy as jnp
from jax.experimental import pallas as pl
from jax.experimental.pallas import tpu as pltpu
from jax.experimental.pallas import tpu_sc as plsc

_D = 2048
_N_TOK = 8192
_TOTAL = _D * _D * 2 + _D * 2
_K = max(1, int(0.05 * _TOTAL))
_TOPBIT = -2147483648

_NC = 2
_NS = 16
_NW = _NC * _NS
_WPS = (_D * _D) // _NW
_CH = 8192
_NCHS = _WPS // _CH
_BW = (2 * _D) // _NW
_NBIN = 4096
_HSLOTS = _NBIN * 16


def _sc_hist_body(shh_ref, sout_ref, sbhh_ref, sbout_ref, out_ref,
                  buf_ref, hist_ref):
    wid = jax.lax.axis_index("s") * _NC + jax.lax.axis_index("c")
    lane = jax.lax.iota(jnp.int32, 16)
    ones = jnp.ones((16,), jnp.int32)

    def zero(i, _):
        hist_ref[pl.ds(i * 16, 16)] = jnp.zeros((16,), jnp.int32)
        return 0

    jax.lax.fori_loop(0, _HSLOTS // 16, zero, 0)

    def eat_block(n, unroll):
        def body(i, _):
            for j in range(unroll):
                v = buf_ref[pl.ds(i * (16 * unroll) + j * 16, 16)]
                b = jax.lax.bitcast_convert_type(v, jnp.int32)
                s = jax.lax.shift_right_arithmetic(b, 20)
                binid = jnp.where(s >= 0, s + 2048, -1 - s)
                plsc.addupdate_scatter(hist_ref, [(binid << 4) | lane], ones)
            return 0

        jax.lax.fori_loop(0, n // (16 * unroll), body, 0)

    def chunks(src_ref):
        def per_chunk(c, _):
            pltpu.sync_copy(src_ref.at[pl.ds(wid * _WPS + c * _CH, _CH)],
                            buf_ref)
            eat_block(_CH, 8)
            return 0

        jax.lax.fori_loop(0, _NCHS, per_chunk, 0)

    chunks(shh_ref)
    chunks(sout_ref)
    for bias_ref in (sbhh_ref, sbout_ref):
        pltpu.sync_copy(bias_ref.at[pl.ds(wid * (_BW // 2), _BW // 2)],
                        buf_ref.at[pl.ds(0, _BW // 2)])
        eat_block(_BW // 2, 4)

    pltpu.sync_copy(hist_ref, out_ref.at[wid])


def _sc_hist(s_hh, s_bhh, s_out, s_bout):
    mesh = plsc.VectorSubcoreMesh(core_axis_name="c", subcore_axis_name="s")
    fn = pl.kernel(
        _sc_hist_body,
        out_type=jax.ShapeDtypeStruct((_NW, _HSLOTS), jnp.int32),
        scratch_types=[pltpu.VMEM((_CH,), jnp.float32),
                       pltpu.VMEM((_HSLOTS,), jnp.int32)],
        mesh=mesh,
        compiler_params=pltpu.CompilerParams(needs_layout_passes=False),
    )
    return fn(s_hh.reshape(_D * _D), s_out.reshape(_D * _D),
              s_bhh, s_bout)


def _finish_body(hist_ref, s_hh_ref, s_bhh_ref, s_out_ref, s_bout_ref,
                 bits_ref):
    acc = hist_ref[pl.ds(0, 512), :]
    for w in range(1, _NW):
        acc = acc + hist_ref[pl.ds(w * 512, 512), :]
    row_i = jax.lax.broadcasted_iota(jnp.int32, (512, 128), 0)
    lane_i = jax.lax.broadcasted_iota(jnp.int32, (512, 128), 1)
    binmat = row_i * 8 + lane_i // 16

    def binstep(i, pb):
        cand = pb | (jnp.int32(1) << (11 - i))
        cnt = jnp.sum(jnp.where(binmat >= cand, acc, 0))
        return jnp.where(cnt >= _K, cand, pb)

    bstar = jax.lax.fori_loop(0, 12, binstep, jnp.int32(0))

    def count_ge(c):
        n = jnp.sum((s_hh_ref[...] >= c).astype(jnp.int32))
        n += jnp.sum((s_out_ref[...] >= c).astype(jnp.int32))
        n += jnp.sum((s_bhh_ref[...] >= c).astype(jnp.int32))
        n += jnp.sum((s_bout_ref[...] >= c).astype(jnp.int32))
        return n

    def _pat_to_f32(p):
        b = jnp.where(p < 0, p ^ jnp.int32(_TOPBIT), jnp.bitwise_not(p))
        return jax.lax.bitcast_convert_type(
            jnp.full((1, 1), b, jnp.int32), jnp.float32)

    def step(i, p):
        shift = 30 - 2 * i
        hits = jnp.int32(0)
        for d in (1, 2, 3):
            cand = p | (jnp.int32(d) << shift)
            cnt = count_ge(_pat_to_f32(cand))
            hits += (cnt >= _K).astype(jnp.int32)
        return p | (hits << shift)

    p = jax.lax.fori_loop(6, 16, step, bstar << 20)
    bits_ref[0, 0] = jnp.where(p < 0, p ^ jnp.int32(_TOPBIT),
                               jnp.bitwise_not(p))


def _select_finish(hists, s_hh, s_bhh, s_out, s_bout):
    vmem = lambda: pl.BlockSpec(memory_space=pltpu.VMEM)
    return pl.pallas_call(
        _finish_body,
        in_specs=[vmem(), vmem(), vmem(), vmem(), vmem()],
        out_specs=pl.BlockSpec(memory_space=pltpu.SMEM),
        out_shape=jax.ShapeDtypeStruct((1, 1), jnp.int32),
    )(hists.reshape(_NW * 512, 128), s_hh, s_bhh.reshape(8, _D // 8),
      s_out, s_bout.reshape(8, _D // 8))


_NCHUNK = 8
_ROWS = _D // _NCHUNK


def _mask_body(thr_ref, whh_ref, shh_ref, wout_ref, sout_ref,
               bhh_ref, sbhh_ref, bout_ref, sbout_ref,
               wmhh_ref, wmout_ref, bmhh_ref, bmout_ref):
    thr = thr_ref[0, 0]
    wmhh_ref[...] = (whh_ref[...] * (shh_ref[...] >= thr)).astype(jnp.bfloat16)
    wmout_ref[...] = (wout_ref[...] * (sout_ref[...] >= thr)
                      ).astype(jnp.bfloat16)
    bmhh_ref[...] = bhh_ref[...] * (sbhh_ref[...] >= thr).astype(jnp.float32)
    bmout_ref[...] = bout_ref[...] * (sbout_ref[...] >= thr).astype(jnp.float32)


def _mask_weights(thr, whh, shh, wout, sout, bhh, sbhh, bout, sbout):
    big = pl.BlockSpec((_ROWS, _D), lambda c: (c, 0))
    vec = pl.BlockSpec((1, _D), lambda c: (0, 0))
    return pl.pallas_call(
        _mask_body,
        grid=(_NCHUNK,),
        in_specs=[pl.BlockSpec(memory_space=pltpu.SMEM),
                  big, big, big, big, vec, vec, vec, vec],
        out_specs=[big, big, vec, vec],
        out_shape=[jax.ShapeDtypeStruct((_D, _D), jnp.bfloat16),
                   jax.ShapeDtypeStruct((_D, _D), jnp.bfloat16),
                   jax.ShapeDtypeStruct((1, _D), jnp.float32),
                   jax.ShapeDtypeStruct((1, _D), jnp.float32)],
    )(thr, whh, shh, wout, sout, bhh, sbhh, bout, sbout)


_TILE_M = 512


def _dot_nt(a, b):
    return jax.lax.dot_general(a, b, (((1,), (1,)), ((), ())),
                               preferred_element_type=jnp.float32)


def _fwd1_body(x_ref, wih_ref, h_ref, wscr_ref):
    @pl.when(pl.program_id(0) == 0)
    def _():
        wscr_ref[...] = wih_ref[...].astype(jnp.bfloat16)

    h_ref[...] = jnp.tanh(_dot_nt(x_ref[...].astype(jnp.bfloat16),
                                  wscr_ref[...])).astype(jnp.bfloat16)


def _fwd1(x, wih):
    xspec = pl.BlockSpec((_TILE_M, _D), lambda m: (m, 0))
    return pl.pallas_call(
        _fwd1_body,
        grid=(_N_TOK // _TILE_M,),
        in_specs=[xspec, pl.BlockSpec((_D, _D), lambda m: (0, 0))],
        out_specs=xspec,
        out_shape=jax.ShapeDtypeStruct((_N_TOK, _D), jnp.bfloat16),
        scratch_shapes=[pltpu.VMEM((_D, _D), jnp.bfloat16)],
    )(x, wih)


def _fwd23_body(h_ref, wmhh_ref, bmhh_ref, wmout_ref, bmout_ref, out_ref):
    h2 = jnp.tanh(_dot_nt(h_ref[...], wmhh_ref[...]) + bmhh_ref[...])
    out_ref[...] = _dot_nt(h2.astype(jnp.bfloat16), wmout_ref[...]) \
        + bmout_ref[...]


def _fwd23(h, wmhh, bmhh, wmout, bmout):
    xspec = pl.BlockSpec((_TILE_M, _D), lambda m: (m, 0))
    wspec = pl.BlockSpec((_D, _D), lambda m: (0, 0))
    vec = pl.BlockSpec((1, _D), lambda m: (0, 0))
    return pl.pallas_call(
        _fwd23_body,
        grid=(_N_TOK // _TILE_M,),
        in_specs=[xspec, wspec, vec, wspec, vec],
        out_specs=xspec,
        out_shape=jax.ShapeDtypeStruct((_N_TOK, _D), jnp.float32),
    )(h, wmhh, bmhh, wmout, bmout)


def kernel(x, W_ih, W_hh, b_hh, W_out, b_out, s_hh, s_b_hh, s_out, s_b_out):
    hists = _sc_hist(s_hh, s_b_hh, s_out, s_b_out)
    h = _fwd1(x, W_ih)
    bits = _select_finish(hists, s_hh, s_b_hh, s_out, s_b_out)
    thr = jax.lax.bitcast_convert_type(bits, jnp.float32)
    wmhh, wmout, bmhh, bmout = _mask_weights(
        thr, W_hh, s_hh, W_out, s_out,
        b_hh.reshape(1, _D), s_b_hh.reshape(1, _D),
        b_out.reshape(1, _D), s_b_out.reshape(1, _D))
    return _fwd23(h, wmhh, bmhh, wmout, bmout)


# speedup vs baseline: 1.2705x; 1.2181x over previous
import functools

import jax
import jax.numpy as jnp
from jax.experimental import pallas as pl
from jax.experimental.pallas import tpu as pltpu
from jax.experimental.pallas import tpu_sc as plsc

_D = 2048
_N_TOK = 8192
_TOTAL = _D * _D * 2 + _D * 2
_K = max(1, int(0.05 * _TOTAL))
_TOPBIT = -2147483648

_NC = 2
_NS = 16
_NW = _NC * _NS
_WPS = (_D * _D) // _NW
_CH = 8192
_NCHS = _WPS // _CH
_BW = (2 * _D) // _NW
_NBIN = 4096
_HSLOTS = _NBIN * 16


def _sc_hist_body(shh_ref, sout_ref, sbhh_ref, sbout_ref, out_ref,
                  buf_ref, hist_ref):
    wid = jax.lax.axis_index("s") * _NC + jax.lax.axis_index("c")
    lane = jax.lax.iota(jnp.int32, 16)
    ones = jnp.ones((16,), jnp.int32)

    def zero(i, _):
        hist_ref[pl.ds(i * 16, 16)] = jnp.zeros((16,), jnp.int32)
        return 0

    jax.lax.fori_loop(0, _HSLOTS // 16, zero, 0)

    def eat_block(n, unroll):
        def body(i, _):
            vs = [buf_ref[pl.ds(i * (16 * unroll) + j * 16, 16)]
                  for j in range(unroll)]
            slots = []
            for v in vs:
                b = jax.lax.bitcast_convert_type(v, jnp.int32)
                s = jax.lax.shift_right_arithmetic(b, 20)
                binid = jnp.where(s >= 0, s + 2048, -1 - s)
                slots.append((binid << 4) | lane)
            for sl in slots:
                plsc.addupdate_scatter(hist_ref, [sl], ones)
            return 0

        jax.lax.fori_loop(0, n // (16 * unroll), body, 0)

    def chunks(src_ref):
        def per_chunk(c, _):
            pltpu.sync_copy(src_ref.at[pl.ds(wid * _WPS + c * _CH, _CH)],
                            buf_ref)
            eat_block(_CH, 8)
            return 0

        jax.lax.fori_loop(0, _NCHS, per_chunk, 0)

    chunks(shh_ref)
    chunks(sout_ref)
    for bias_ref in (sbhh_ref, sbout_ref):
        pltpu.sync_copy(bias_ref.at[pl.ds(wid * (_BW // 2), _BW // 2)],
                        buf_ref.at[pl.ds(0, _BW // 2)])
        eat_block(_BW // 2, 4)

    pltpu.sync_copy(hist_ref, out_ref.at[wid])


def _sc_hist(s_hh, s_bhh, s_out, s_bout):
    mesh = plsc.VectorSubcoreMesh(core_axis_name="c", subcore_axis_name="s")
    fn = pl.kernel(
        _sc_hist_body,
        out_type=jax.ShapeDtypeStruct((_NW, _HSLOTS), jnp.int32),
        scratch_types=[pltpu.VMEM((_CH,), jnp.float32),
                       pltpu.VMEM((_HSLOTS,), jnp.int32)],
        mesh=mesh,
        compiler_params=pltpu.CompilerParams(needs_layout_passes=False),
    )
    return fn(s_hh.reshape(_D * _D), s_out.reshape(_D * _D),
              s_bhh, s_bout)


def _finish_body(hist_ref, s_hh_ref, s_bhh_ref, s_out_ref, s_bout_ref,
                 bits_ref):
    acc = hist_ref[pl.ds(0, 512), :]
    for w in range(1, _NW):
        acc = acc + hist_ref[pl.ds(w * 512, 512), :]
    row_i = jax.lax.broadcasted_iota(jnp.int32, (512, 128), 0)
    lane_i = jax.lax.broadcasted_iota(jnp.int32, (512, 128), 1)
    binmat = row_i * 8 + lane_i // 16

    def binstep(i, pb):
        cand = pb | (jnp.int32(1) << (11 - i))
        cnt = jnp.sum(jnp.where(binmat >= cand, acc, 0))
        return jnp.where(cnt >= _K, cand, pb)

    bstar = jax.lax.fori_loop(0, 12, binstep, jnp.int32(0))

    def count_ge(c):
        n = jnp.sum((s_hh_ref[...] >= c).astype(jnp.int32))
        n += jnp.sum((s_out_ref[...] >= c).astype(jnp.int32))
        n += jnp.sum((s_bhh_ref[...] >= c).astype(jnp.int32))
        n += jnp.sum((s_bout_ref[...] >= c).astype(jnp.int32))
        return n

    def _pat_to_f32(p):
        b = jnp.where(p < 0, p ^ jnp.int32(_TOPBIT), jnp.bitwise_not(p))
        return jax.lax.bitcast_convert_type(
            jnp.full((1, 1), b, jnp.int32), jnp.float32)

    def step(i, p):
        shift = 30 - 2 * i
        hits = jnp.int32(0)
        for d in (1, 2, 3):
            cand = p | (jnp.int32(d) << shift)
            cnt = count_ge(_pat_to_f32(cand))
            hits += (cnt >= _K).astype(jnp.int32)
        return p | (hits << shift)

    p = jax.lax.fori_loop(6, 16, step, bstar << 20)
    bits_ref[0, 0] = jnp.where(p < 0, p ^ jnp.int32(_TOPBIT),
                               jnp.bitwise_not(p))


def _select_finish(hists, s_hh, s_bhh, s_out, s_bout):
    vmem = lambda: pl.BlockSpec(memory_space=pltpu.VMEM)
    return pl.pallas_call(
        _finish_body,
        in_specs=[vmem(), vmem(), vmem(), vmem(), vmem()],
        out_specs=pl.BlockSpec(memory_space=pltpu.SMEM),
        out_shape=jax.ShapeDtypeStruct((1, 1), jnp.int32),
    )(hists.reshape(_NW * 512, 128), s_hh, s_bhh.reshape(8, _D // 8),
      s_out, s_bout.reshape(8, _D // 8))


_NCHUNK = 8
_ROWS = _D // _NCHUNK


def _mask_body(thr_ref, whh_ref, shh_ref, wout_ref, sout_ref,
               bhh_ref, sbhh_ref, bout_ref, sbout_ref,
               wmhh_ref, wmout_ref, bmhh_ref, bmout_ref):
    thr = thr_ref[0, 0]
    wmhh_ref[...] = (whh_ref[...] * (shh_ref[...] >= thr)).astype(jnp.bfloat16)
    wmout_ref[...] = (wout_ref[...] * (sout_ref[...] >= thr)
                      ).astype(jnp.bfloat16)
    bmhh_ref[...] = bhh_ref[...] * (sbhh_ref[...] >= thr).astype(jnp.float32)
    bmout_ref[...] = bout_ref[...] * (sbout_ref[...] >= thr).astype(jnp.float32)


def _mask_weights(thr, whh, shh, wout, sout, bhh, sbhh, bout, sbout):
    big = pl.BlockSpec((_ROWS, _D), lambda c: (c, 0))
    vec = pl.BlockSpec((1, _D), lambda c: (0, 0))
    return pl.pallas_call(
        _mask_body,
        grid=(_NCHUNK,),
        in_specs=[pl.BlockSpec(memory_space=pltpu.SMEM),
                  big, big, big, big, vec, vec, vec, vec],
        out_specs=[big, big, vec, vec],
        out_shape=[jax.ShapeDtypeStruct((_D, _D), jnp.bfloat16),
                   jax.ShapeDtypeStruct((_D, _D), jnp.bfloat16),
                   jax.ShapeDtypeStruct((1, _D), jnp.float32),
                   jax.ShapeDtypeStruct((1, _D), jnp.float32)],
    )(thr, whh, shh, wout, sout, bhh, sbhh, bout, sbout)


_TILE_M = 512


def _dot_nt(a, b):
    return jax.lax.dot_general(a, b, (((1,), (1,)), ((), ())),
                               preferred_element_type=jnp.float32)


def _fwd1_body(x_ref, wih_ref, h_ref, wscr_ref):
    @pl.when(pl.program_id(0) == 0)
    def _():
        wscr_ref[...] = wih_ref[...].astype(jnp.bfloat16)

    h_ref[...] = jnp.tanh(_dot_nt(x_ref[...].astype(jnp.bfloat16),
                                  wscr_ref[...])).astype(jnp.bfloat16)


def _fwd1(x, wih):
    xspec = pl.BlockSpec((_TILE_M, _D), lambda m: (m, 0))
    return pl.pallas_call(
        _fwd1_body,
        grid=(_N_TOK // _TILE_M,),
        in_specs=[xspec, pl.BlockSpec((_D, _D), lambda m: (0, 0))],
        out_specs=xspec,
        out_shape=jax.ShapeDtypeStruct((_N_TOK, _D), jnp.bfloat16),
        scratch_shapes=[pltpu.VMEM((_D, _D), jnp.bfloat16)],
    )(x, wih)


def _fwd23_body(h_ref, wmhh_ref, bmhh_ref, wmout_ref, bmout_ref, out_ref):
    h2 = jnp.tanh(_dot_nt(h_ref[...], wmhh_ref[...]) + bmhh_ref[...])
    out_ref[...] = _dot_nt(h2.astype(jnp.bfloat16), wmout_ref[...]) \
        + bmout_ref[...]


def _fwd23(h, wmhh, bmhh, wmout, bmout):
    xspec = pl.BlockSpec((_TILE_M, _D), lambda m: (m, 0))
    wspec = pl.BlockSpec((_D, _D), lambda m: (0, 0))
    vec = pl.BlockSpec((1, _D), lambda m: (0, 0))
    return pl.pallas_call(
        _fwd23_body,
        grid=(_N_TOK // _TILE_M,),
        in_specs=[xspec, wspec, vec, wspec, vec],
        out_specs=xspec,
        out_shape=jax.ShapeDtypeStruct((_N_TOK, _D), jnp.float32),
    )(h, wmhh, bmhh, wmout, bmout)


def kernel(x, W_ih, W_hh, b_hh, W_out, b_out, s_hh, s_b_hh, s_out, s_b_out):
    hists = _sc_hist(s_hh, s_b_hh, s_out, s_b_out)
    h = _fwd1(x, W_ih)
    bits = _select_finish(hists, s_hh, s_b_hh, s_out, s_b_out)
    thr = jax.lax.bitcast_convert_type(bits, jnp.float32)
    wmhh, wmout, bmhh, bmout = _mask_weights(
        thr, W_hh, s_hh, W_out, s_out,
        b_hh.reshape(1, _D), s_b_hh.reshape(1, _D),
        b_out.reshape(1, _D), s_b_out.reshape(1, _D))
    return _fwd23(h, wmhh, bmhh, wmout, bmout)


# speedup vs baseline: 1.3013x; 1.0242x over previous
import functools

import jax
import jax.numpy as jnp
from jax.experimental import pallas as pl
from jax.experimental.pallas import tpu as pltpu
from jax.experimental.pallas import tpu_sc as plsc

_D = 2048
_N_TOK = 8192
_TOTAL = _D * _D * 2 + _D * 2
_K = max(1, int(0.05 * _TOTAL))
_TOPBIT = -2147483648

_NC = 2
_NS = 16
_NW = _NC * _NS
_WPS = (_D * _D) // _NW
_CH = 8192
_NCHS = _WPS // _CH
_BW = (2 * _D) // _NW
_NBIN = 4096
_HSLOTS = _NBIN * 16


def _sc_hist_body(shh_ref, sout_ref, sbhh_ref, sbout_ref, out_ref,
                  buf_ref, hist_ref):
    wid = jax.lax.axis_index("s") * _NC + jax.lax.axis_index("c")
    lane = jax.lax.iota(jnp.int32, 16)
    ones = jnp.ones((16,), jnp.int32)

    def zero(i, _):
        hist_ref[pl.ds(i * 16, 16)] = jnp.zeros((16,), jnp.int32)
        return 0

    jax.lax.fori_loop(0, _HSLOTS // 16, zero, 0)

    def eat_block(n, unroll):
        def body(i, _):
            vs = [buf_ref[pl.ds(i * (16 * unroll) + j * 16, 16)]
                  for j in range(unroll)]
            slots = []
            for v in vs:
                b = jax.lax.bitcast_convert_type(v, jnp.int32)
                s = jax.lax.shift_right_arithmetic(b, 20)
                binid = jnp.where(s >= 0, s + 2048, -1 - s)
                slots.append((binid << 4) | lane)
            for sl in slots:
                plsc.addupdate_scatter(hist_ref, [sl], ones)
            return 0

        jax.lax.fori_loop(0, n // (16 * unroll), body, 0)

    def chunks(src_ref):
        def per_chunk(c, _):
            pltpu.sync_copy(src_ref.at[pl.ds(wid * _WPS + c * _CH, _CH)],
                            buf_ref)
            eat_block(_CH, 8)
            return 0

        jax.lax.fori_loop(0, _NCHS, per_chunk, 0)

    chunks(shh_ref)
    chunks(sout_ref)
    for bias_ref in (sbhh_ref, sbout_ref):
        pltpu.sync_copy(bias_ref.at[pl.ds(wid * (_BW // 2), _BW // 2)],
                        buf_ref.at[pl.ds(0, _BW // 2)])
        eat_block(_BW // 2, 4)

    pltpu.sync_copy(hist_ref, out_ref.at[wid])


def _sc_hist(s_hh, s_bhh, s_out, s_bout):
    mesh = plsc.VectorSubcoreMesh(core_axis_name="c", subcore_axis_name="s")
    fn = pl.kernel(
        _sc_hist_body,
        out_type=jax.ShapeDtypeStruct((_NW, _HSLOTS), jnp.int32),
        scratch_types=[pltpu.VMEM((_CH,), jnp.float32),
                       pltpu.VMEM((_HSLOTS,), jnp.int32)],
        mesh=mesh,
        compiler_params=pltpu.CompilerParams(needs_layout_passes=False),
    )
    return fn(s_hh.reshape(_D * _D), s_out.reshape(_D * _D),
              s_bhh, s_bout)


def _finish_mask_body(hist_ref, s_hh_ref, s_bhh_ref, s_out_ref, s_bout_ref,
                      whh_ref, wout_ref, bhh_ref, sbhh2_ref, bout_ref,
                      sbout2_ref, wmhh_ref, wmout_ref, bmhh_ref, bmout_ref,
                      pbits_ref):
    c = pl.program_id(0)

    @pl.when(c == 0)
    def _():
        _finish_select(hist_ref, s_hh_ref, s_bhh_ref, s_out_ref, s_bout_ref,
                       pbits_ref)

    @pl.when(c > 0)
    def _():
        thr = jax.lax.bitcast_convert_type(
            jnp.full((1, 1), pbits_ref[0, 0], jnp.int32), jnp.float32)
        r = (c - 1) * _ROWS
        shh = s_hh_ref[pl.ds(r, _ROWS), :]
        sout = s_out_ref[pl.ds(r, _ROWS), :]
        wmhh_ref[...] = (whh_ref[...] * (shh >= thr)).astype(jnp.bfloat16)
        wmout_ref[...] = (wout_ref[...] * (sout >= thr)).astype(jnp.bfloat16)

        @pl.when(c == 1)
        def _():
            bmhh_ref[...] = bhh_ref[...] * (sbhh2_ref[...] >= thr)
            bmout_ref[...] = bout_ref[...] * (sbout2_ref[...] >= thr)


def _finish_select(hist_ref, s_hh_ref, s_bhh_ref, s_out_ref, s_bout_ref,
                   bits_ref):
    acc = hist_ref[pl.ds(0, 512), :]
    for w in range(1, _NW):
        acc = acc + hist_ref[pl.ds(w * 512, 512), :]
    row_i = jax.lax.broadcasted_iota(jnp.int32, (512, 128), 0)
    lane_i = jax.lax.broadcasted_iota(jnp.int32, (512, 128), 1)
    binmat = row_i * 8 + lane_i // 16

    def binstep(i, pb):
        cand = pb | (jnp.int32(1) << (11 - i))
        cnt = jnp.sum(jnp.where(binmat >= cand, acc, 0))
        return jnp.where(cnt >= _K, cand, pb)

    bstar = jax.lax.fori_loop(0, 12, binstep, jnp.int32(0))

    def count_ge(c):
        n = jnp.sum((s_hh_ref[...] >= c).astype(jnp.int32))
        n += jnp.sum((s_out_ref[...] >= c).astype(jnp.int32))
        n += jnp.sum((s_bhh_ref[...] >= c).astype(jnp.int32))
        n += jnp.sum((s_bout_ref[...] >= c).astype(jnp.int32))
        return n

    def _pat_to_f32(p):
        b = jnp.where(p < 0, p ^ jnp.int32(_TOPBIT), jnp.bitwise_not(p))
        return jax.lax.bitcast_convert_type(
            jnp.full((1, 1), b, jnp.int32), jnp.float32)

    def step(i, p):
        shift = 30 - 2 * i
        hits = jnp.int32(0)
        for d in (1, 2, 3):
            cand = p | (jnp.int32(d) << shift)
            cnt = count_ge(_pat_to_f32(cand))
            hits += (cnt >= _K).astype(jnp.int32)
        return p | (hits << shift)

    p = jax.lax.fori_loop(6, 16, step, bstar << 20)
    bits_ref[0, 0] = jnp.where(p < 0, p ^ jnp.int32(_TOPBIT),
                               jnp.bitwise_not(p))


_NCHUNK = 8
_ROWS = _D // _NCHUNK


def _finish_mask(hists, s_hh, s_bhh, s_out, s_bout, whh, wout,
                 bhh, bout):
    vmem = lambda: pl.BlockSpec(memory_space=pltpu.VMEM)
    res = lambda: pl.BlockSpec((_NW * 512, 128), lambda c: (0, 0))
    sres = lambda: pl.BlockSpec((_D, _D), lambda c: (0, 0))
    sb = lambda: pl.BlockSpec((8, _D // 8), lambda c: (0, 0))
    big = pl.BlockSpec((_ROWS, _D), lambda c: (jnp.maximum(c - 1, 0), 0))
    vec = pl.BlockSpec((1, _D), lambda c: (0, 0))
    return pl.pallas_call(
        _finish_mask_body,
        grid=(1 + _NCHUNK,),
        in_specs=[res(), sres(), sb(), sres(), sb(),
                  big, big, vec, vec, vec, vec],
        out_specs=[big, big, vec, vec],
        out_shape=[jax.ShapeDtypeStruct((_D, _D), jnp.bfloat16),
                   jax.ShapeDtypeStruct((_D, _D), jnp.bfloat16),
                   jax.ShapeDtypeStruct((1, _D), jnp.float32),
                   jax.ShapeDtypeStruct((1, _D), jnp.float32)],
        scratch_shapes=[pltpu.SMEM((1, 1), jnp.int32)],
    )(hists.reshape(_NW * 512, 128), s_hh, s_bhh.reshape(8, _D // 8),
      s_out, s_bout.reshape(8, _D // 8), whh, wout,
      bhh.reshape(1, _D), s_bhh.reshape(1, _D),
      bout.reshape(1, _D), s_bout.reshape(1, _D))


_TILE_M = 512


def _dot_nt(a, b):
    return jax.lax.dot_general(a, b, (((1,), (1,)), ((), ())),
                               preferred_element_type=jnp.float32)


def _fwd1_body(x_ref, wih_ref, h_ref, wscr_ref):
    @pl.when(pl.program_id(0) == 0)
    def _():
        wscr_ref[...] = wih_ref[...].astype(jnp.bfloat16)

    h_ref[...] = jnp.tanh(_dot_nt(x_ref[...].astype(jnp.bfloat16),
                                  wscr_ref[...])).astype(jnp.bfloat16)


def _fwd1(x, wih):
    xspec = pl.BlockSpec((_TILE_M, _D), lambda m: (m, 0))
    return pl.pallas_call(
        _fwd1_body,
        grid=(_N_TOK // _TILE_M,),
        in_specs=[xspec, pl.BlockSpec((_D, _D), lambda m: (0, 0))],
        out_specs=xspec,
        out_shape=jax.ShapeDtypeStruct((_N_TOK, _D), jnp.bfloat16),
        scratch_shapes=[pltpu.VMEM((_D, _D), jnp.bfloat16)],
    )(x, wih)


def _fwd23_body(h_ref, wmhh_ref, bmhh_ref, wmout_ref, bmout_ref, out_ref):
    h2 = jnp.tanh(_dot_nt(h_ref[...], wmhh_ref[...]) + bmhh_ref[...])
    out_ref[...] = _dot_nt(h2.astype(jnp.bfloat16), wmout_ref[...]) \
        + bmout_ref[...]


def _fwd23(h, wmhh, bmhh, wmout, bmout):
    xspec = pl.BlockSpec((_TILE_M, _D), lambda m: (m, 0))
    wspec = pl.BlockSpec((_D, _D), lambda m: (0, 0))
    vec = pl.BlockSpec((1, _D), lambda m: (0, 0))
    return pl.pallas_call(
        _fwd23_body,
        grid=(_N_TOK // _TILE_M,),
        in_specs=[xspec, wspec, vec, wspec, vec],
        out_specs=xspec,
        out_shape=jax.ShapeDtypeStruct((_N_TOK, _D), jnp.float32),
    )(h, wmhh, bmhh, wmout, bmout)


def kernel(x, W_ih, W_hh, b_hh, W_out, b_out, s_hh, s_b_hh, s_out, s_b_out):
    hists = _sc_hist(s_hh, s_b_hh, s_out, s_b_out)
    h = _fwd1(x, W_ih)
    wmhh, wmout, bmhh, bmout = _finish_mask(
        hists, s_hh, s_b_hh, s_out, s_b_out, W_hh, W_out, b_hh, b_out)
    return _fwd23(h, wmhh, bmhh, wmout, bmout)


# speedup vs baseline: 1.3431x; 1.0321x over previous
import functools

import jax
import jax.numpy as jnp
from jax.experimental import pallas as pl
from jax.experimental.pallas import tpu as pltpu
from jax.experimental.pallas import tpu_sc as plsc

_D = 2048
_N_TOK = 8192
_TOTAL = _D * _D * 2 + _D * 2
_K = max(1, int(0.05 * _TOTAL))
_TOPBIT = -2147483648

_NC = 2
_NS = 16
_NW = _NC * _NS
_WPS = (_D * _D) // _NW
_CH = 8192
_NCHS = _WPS // _CH
_BW = (2 * _D) // _NW
_NBIN = 4096
_HSLOTS = _NBIN * 16


def _sc_hist_body(shh_ref, sout_ref, sbhh_ref, sbout_ref, out_ref,
                  buf_ref, hist_ref):
    wid = jax.lax.axis_index("s") * _NC + jax.lax.axis_index("c")
    lane = jax.lax.iota(jnp.int32, 16)
    ones = jnp.ones((16,), jnp.int32)

    zeros16 = jnp.zeros((16,), jnp.int32)

    def zero(i, _):
        for j in range(16):
            hist_ref[pl.ds(i * 256 + j * 16, 16)] = zeros16
        return 0

    jax.lax.fori_loop(0, _HSLOTS // 256, zero, 0)

    def eat_block(n, unroll):
        def body(i, _):
            vs = [buf_ref[pl.ds(i * (16 * unroll) + j * 16, 16)]
                  for j in range(unroll)]
            slots = []
            for v in vs:
                b = jax.lax.bitcast_convert_type(v, jnp.int32)
                s = jax.lax.shift_right_arithmetic(b, 20)
                binid = jnp.where(s >= 0, s + 2048, -1 - s)
                slots.append((binid << 4) | lane)
            for sl in slots:
                plsc.addupdate_scatter(hist_ref, [sl], ones)
            return 0

        jax.lax.fori_loop(0, n // (16 * unroll), body, 0)

    def chunks(src_ref):
        def per_chunk(c, _):
            pltpu.sync_copy(src_ref.at[pl.ds(wid * _WPS + c * _CH, _CH)],
                            buf_ref)
            eat_block(_CH, 8)
            return 0

        jax.lax.fori_loop(0, _NCHS, per_chunk, 0)

    chunks(shh_ref)
    chunks(sout_ref)
    for bias_ref in (sbhh_ref, sbout_ref):
        pltpu.sync_copy(bias_ref.at[pl.ds(wid * (_BW // 2), _BW // 2)],
                        buf_ref.at[pl.ds(0, _BW // 2)])
        eat_block(_BW // 2, 4)

    pltpu.sync_copy(hist_ref, out_ref.at[wid])


def _sc_hist(s_hh, s_bhh, s_out, s_bout):
    mesh = plsc.VectorSubcoreMesh(core_axis_name="c", subcore_axis_name="s")
    fn = pl.kernel(
        _sc_hist_body,
        out_type=jax.ShapeDtypeStruct((_NW, _HSLOTS), jnp.int32),
        scratch_types=[pltpu.VMEM((_CH,), jnp.float32),
                       pltpu.VMEM((_HSLOTS,), jnp.int32)],
        mesh=mesh,
        compiler_params=pltpu.CompilerParams(needs_layout_passes=False),
    )
    return fn(s_hh.reshape(_D * _D), s_out.reshape(_D * _D),
              s_bhh, s_bout)


def _finish_mask_body(hist_ref, s_hh_ref, s_bhh_ref, s_out_ref, s_bout_ref,
                      whh_ref, wout_ref, bhh_ref, sbhh2_ref, bout_ref,
                      sbout2_ref, wmhh_ref, wmout_ref, bmhh_ref, bmout_ref,
                      pbits_ref):
    c = pl.program_id(0)

    @pl.when(c == 0)
    def _():
        _finish_select(hist_ref, s_hh_ref, s_bhh_ref, s_out_ref, s_bout_ref,
                       pbits_ref)

    @pl.when(c > 0)
    def _():
        thr = jax.lax.bitcast_convert_type(
            jnp.full((1, 1), pbits_ref[0, 0], jnp.int32), jnp.float32)
        r = (c - 1) * _ROWS
        shh = s_hh_ref[pl.ds(r, _ROWS), :]
        sout = s_out_ref[pl.ds(r, _ROWS), :]
        wmhh_ref[...] = (whh_ref[...] * (shh >= thr)).astype(jnp.bfloat16)
        wmout_ref[...] = (wout_ref[...] * (sout >= thr)).astype(jnp.bfloat16)

        @pl.when(c == 1)
        def _():
            bmhh_ref[...] = bhh_ref[...] * (sbhh2_ref[...] >= thr)
            bmout_ref[...] = bout_ref[...] * (sbout2_ref[...] >= thr)


def _finish_select(hist_ref, s_hh_ref, s_bhh_ref, s_out_ref, s_bout_ref,
                   bits_ref):
    acc = hist_ref[pl.ds(0, 512), :]
    for w in range(1, _NW):
        acc = acc + hist_ref[pl.ds(w * 512, 512), :]
    row_i = jax.lax.broadcasted_iota(jnp.int32, (512, 128), 0)
    lane_i = jax.lax.broadcasted_iota(jnp.int32, (512, 128), 1)
    binmat = row_i * 8 + lane_i // 16

    def binstep(i, pb):
        cand = pb | (jnp.int32(1) << (11 - i))
        cnt = jnp.sum(jnp.where(binmat >= cand, acc, 0))
        return jnp.where(cnt >= _K, cand, pb)

    bstar = jax.lax.fori_loop(0, 12, binstep, jnp.int32(0))

    def count_ge(c):
        n = jnp.sum((s_hh_ref[...] >= c).astype(jnp.int32))
        n += jnp.sum((s_out_ref[...] >= c).astype(jnp.int32))
        n += jnp.sum((s_bhh_ref[...] >= c).astype(jnp.int32))
        n += jnp.sum((s_bout_ref[...] >= c).astype(jnp.int32))
        return n

    def _pat_to_f32(p):
        b = jnp.where(p < 0, p ^ jnp.int32(_TOPBIT), jnp.bitwise_not(p))
        return jax.lax.bitcast_convert_type(
            jnp.full((1, 1), b, jnp.int32), jnp.float32)

    def step(i, p):
        shift = 30 - 2 * i
        hits = jnp.int32(0)
        for d in (1, 2, 3):
            cand = p | (jnp.int32(d) << shift)
            cnt = count_ge(_pat_to_f32(cand))
            hits += (cnt >= _K).astype(jnp.int32)
        return p | (hits << shift)

    p = jax.lax.fori_loop(6, 16, step, bstar << 20)
    bits_ref[0, 0] = jnp.where(p < 0, p ^ jnp.int32(_TOPBIT),
                               jnp.bitwise_not(p))


_NCHUNK = 8
_ROWS = _D // _NCHUNK


def _finish_mask(hists, s_hh, s_bhh, s_out, s_bout, whh, wout,
                 bhh, bout):
    vmem = lambda: pl.BlockSpec(memory_space=pltpu.VMEM)
    res = lambda: pl.BlockSpec((_NW * 512, 128), lambda c: (0, 0))
    sres = lambda: pl.BlockSpec((_D, _D), lambda c: (0, 0))
    sb = lambda: pl.BlockSpec((8, _D // 8), lambda c: (0, 0))
    big = pl.BlockSpec((_ROWS, _D), lambda c: (jnp.maximum(c - 1, 0), 0))
    vec = pl.BlockSpec((1, _D), lambda c: (0, 0))
    return pl.pallas_call(
        _finish_mask_body,
        grid=(1 + _NCHUNK,),
        in_specs=[res(), sres(), sb(), sres(), sb(),
                  big, big, vec, vec, vec, vec],
        out_specs=[big, big, vec, vec],
        out_shape=[jax.ShapeDtypeStruct((_D, _D), jnp.bfloat16),
                   jax.ShapeDtypeStruct((_D, _D), jnp.bfloat16),
                   jax.ShapeDtypeStruct((1, _D), jnp.float32),
                   jax.ShapeDtypeStruct((1, _D), jnp.float32)],
        scratch_shapes=[pltpu.SMEM((1, 1), jnp.int32)],
    )(hists.reshape(_NW * 512, 128), s_hh, s_bhh.reshape(8, _D // 8),
      s_out, s_bout.reshape(8, _D // 8), whh, wout,
      bhh.reshape(1, _D), s_bhh.reshape(1, _D),
      bout.reshape(1, _D), s_bout.reshape(1, _D))


_TILE_M = 512


def _dot_nt(a, b):
    return jax.lax.dot_general(a, b, (((1,), (1,)), ((), ())),
                               preferred_element_type=jnp.float32)


def _fwd1_body(x_ref, wih_ref, h_ref, wscr_ref):
    @pl.when(pl.program_id(0) == 0)
    def _():
        wscr_ref[...] = wih_ref[...].astype(jnp.bfloat16)

    h_ref[...] = jnp.tanh(_dot_nt(x_ref[...].astype(jnp.bfloat16),
                                  wscr_ref[...])).astype(jnp.bfloat16)


def _fwd1(x, wih):
    xspec = pl.BlockSpec((_TILE_M, _D), lambda m: (m, 0))
    return pl.pallas_call(
        _fwd1_body,
        grid=(_N_TOK // _TILE_M,),
        in_specs=[xspec, pl.BlockSpec((_D, _D), lambda m: (0, 0))],
        out_specs=xspec,
        out_shape=jax.ShapeDtypeStruct((_N_TOK, _D), jnp.bfloat16),
        scratch_shapes=[pltpu.VMEM((_D, _D), jnp.bfloat16)],
    )(x, wih)


def _fwd23_body(h_ref, wmhh_ref, bmhh_ref, wmout_ref, bmout_ref, out_ref):
    h2 = jnp.tanh(_dot_nt(h_ref[...], wmhh_ref[...]) + bmhh_ref[...])
    out_ref[...] = _dot_nt(h2.astype(jnp.bfloat16), wmout_ref[...]) \
        + bmout_ref[...]


def _fwd23(h, wmhh, bmhh, wmout, bmout):
    xspec = pl.BlockSpec((_TILE_M, _D), lambda m: (m, 0))
    wspec = pl.BlockSpec((_D, _D), lambda m: (0, 0))
    vec = pl.BlockSpec((1, _D), lambda m: (0, 0))
    return pl.pallas_call(
        _fwd23_body,
        grid=(_N_TOK // _TILE_M,),
        in_specs=[xspec, wspec, vec, wspec, vec],
        out_specs=xspec,
        out_shape=jax.ShapeDtypeStruct((_N_TOK, _D), jnp.float32),
    )(h, wmhh, bmhh, wmout, bmout)


def kernel(x, W_ih, W_hh, b_hh, W_out, b_out, s_hh, s_b_hh, s_out, s_b_out):
    hists = _sc_hist(s_hh, s_b_hh, s_out, s_b_out)
    h = _fwd1(x, W_ih)
    wmhh, wmout, bmhh, bmout = _finish_mask(
        hists, s_hh, s_b_hh, s_out, s_b_out, W_hh, W_out, b_hh, b_out)
    return _fwd23(h, wmhh, bmhh, wmout, bmout)
